# Initial kernel scaffold; baseline (speedup 1.0000x reference)
import jax
import jax.numpy as jnp
from jax.experimental import pallas as pl


def kernel(tile_x, tile_yx, tile_t, token_x, token_t, cursor_fine_yxp, cursor_coarse_yx, cursor_t, readout_x, readout_t, tile_W, tile_b, token_table, cursor_fine_table, readout_table, spatial_pe, temporal_pe, g_tile, b_tile, g_token, b_token, g_cursor, b_cursor, g_readout, b_readout, g_spatial, b_spatial, g_temporal, b_temporal):
    raise NotImplementedError("write your pallas kernel here")


# speedup vs baseline: 1.5337x; 1.5337x over previous
import functools

import jax
import jax.numpy as jnp
from jax import lax
from jax.experimental import pallas as pl
from jax.experimental.pallas import tpu as pltpu
from jax.experimental.pallas import tpu_sc as plsc

B, L, C = 1024, 20, 128
N = B * L
TPIX = 768
NC, NS = 2, 16
NW = NC * NS
PW = N // NW
CH = 128
NCHUNK = PW // CH
NCP = 8
EPS = 1e-5


def _norm_tables_body(sp, te, cf, ro, gs, bs, gt, bt, gc, bc, gr, br,
                      osp, ote, ocf, oro):
    def ln(x_ref, g_ref, b_ref, o_ref):
        x = x_ref[...]
        m = jnp.mean(x, axis=-1, keepdims=True)
        v = jnp.mean((x - m) ** 2, axis=-1, keepdims=True)
        o_ref[...] = (x - m) / jnp.sqrt(v + EPS) * g_ref[...][None, :] \
            + b_ref[...][None, :]
    ln(sp, gs, bs, osp)
    ln(te, gt, bt, ote)
    ln(cf, gc, bc, ocf)
    ln(ro, gr, br, oro)


def _norm_tables(spatial_pe, temporal_pe, cursor_fine_table, readout_table,
                 g_spatial, b_spatial, g_temporal, b_temporal,
                 g_cursor, b_cursor, g_readout, b_readout):
    outs = [jax.ShapeDtypeStruct(t.shape, jnp.float32)
            for t in (spatial_pe, temporal_pe, cursor_fine_table,
                      readout_table)]
    return pl.pallas_call(_norm_tables_body, out_shape=outs)(
        spatial_pe, temporal_pe, cursor_fine_table, readout_table,
        g_spatial, b_spatial, g_temporal, b_temporal,
        g_cursor, b_cursor, g_readout, b_readout)


def _sc_gather_body(sg, tg, cg, rg, tok,
                    ix_tile_yx, ix_tile_t, ix_token_x, ix_token_t,
                    ix_fine, ix_coarse, ix_cur_t, ix_ro_x, ix_ro_t,
                    pe_tile, tok_raw, pe_tok, cur_e, ro_e,
                    v_tile_yx, v_tile_t, v_token_x, v_token_t,
                    v_fine, v_coarse, v_cur_t, v_ro_x, v_ro_t,
                    bufa, bufb, sema, semb):
    wid = lax.axis_index("s") * NC + lax.axis_index("c")
    ibase = wid * NCP

    for hbm, vm in ((ix_tile_yx, v_tile_yx), (ix_tile_t, v_tile_t),
                    (ix_token_x, v_token_x), (ix_token_t, v_token_t),
                    (ix_fine, v_fine), (ix_coarse, v_coarse),
                    (ix_cur_t, v_cur_t), (ix_ro_x, v_ro_x),
                    (ix_ro_t, v_ro_t)):
        pltpu.sync_copy(hbm.at[pl.ds(ibase, NCP)], vm)

    def chunk(j, carry):
        out_off = wid * PW + j * CH

        def gather(tbl, idx_vmem, buf, sem):
            pltpu.async_copy(tbl.at[idx_vmem.at[j]], buf, sem).wait()

        def acc(i, _):
            for c in range(8):
                s = (i, pl.ds(c * 16, 16))
                bufa[s] = bufa[s] + bufb[s]
            return 0

        def add_and_store(out_ref):
            lax.fori_loop(0, CH, acc, 0)
            pltpu.sync_copy(bufa, out_ref.at[pl.ds(out_off, CH)])

        gather(sg, v_tile_yx, bufa, sema)
        gather(tg, v_tile_t, bufb, semb)
        add_and_store(pe_tile)
        gather(tok, v_token_x, bufa, sema)
        pltpu.sync_copy(bufa, tok_raw.at[pl.ds(out_off, CH)])
        gather(tg, v_token_t, bufa, sema)
        pltpu.sync_copy(bufa, pe_tok.at[pl.ds(out_off, CH)])
        gather(cg, v_fine, bufa, sema)
        gather(sg, v_coarse, bufb, semb)
        lax.fori_loop(0, CH, acc, 0)
        gather(tg, v_cur_t, bufb, semb)
        add_and_store(cur_e)
        gather(rg, v_ro_x, bufa, sema)
        gather(tg, v_ro_t, bufb, semb)
        add_and_store(ro_e)
        return carry

    lax.fori_loop(0, NCHUNK, chunk, 0)


def _sc_gather(sg, tg, cg, rg, tok, idx_list):
    mesh = plsc.VectorSubcoreMesh(core_axis_name="c", subcore_axis_name="s")
    f32 = jnp.float32
    out_type = [jax.ShapeDtypeStruct((N, C), f32) for _ in range(5)]
    idx_vmem = [pltpu.VMEM((NCP, CH), jnp.int32) for _ in range(9)]
    scratch = idx_vmem + [
        pltpu.VMEM((CH, C), f32), pltpu.VMEM((CH, C), f32),
        pltpu.SemaphoreType.DMA, pltpu.SemaphoreType.DMA,
    ]
    run = pl.kernel(_sc_gather_body, out_type=out_type, mesh=mesh,
                    scratch_types=scratch)
    return run(sg, tg, cg, rg, tok, *idx_list)


def _tile_matmul_body(x_ref, w_ref, bias_ref, g_ref, b_ref, o_ref):
    y = jnp.dot(x_ref[...], w_ref[...], preferred_element_type=jnp.float32)
    y = y + bias_ref[...][None, :]
    m = jnp.mean(y, axis=-1, keepdims=True)
    v = jnp.mean((y - m) ** 2, axis=-1, keepdims=True)
    o_ref[...] = (y - m) / jnp.sqrt(v + EPS) * g_ref[...][None, :] \
        + b_ref[...][None, :]


def _tile_matmul(x2, tile_W, tile_b, g_tile, b_tile):
    RB = 2560
    grid = (N // RB,)
    return pl.pallas_call(
        _tile_matmul_body,
        grid=grid,
        in_specs=[
            pl.BlockSpec((RB, TPIX), lambda i: (i, 0)),
            pl.BlockSpec((TPIX, C), lambda i: (0, 0)),
            pl.BlockSpec((C,), lambda i: (0,)),
            pl.BlockSpec((C,), lambda i: (0,)),
            pl.BlockSpec((C,), lambda i: (0,)),
        ],
        out_specs=pl.BlockSpec((RB, C), lambda i: (i, 0)),
        out_shape=jax.ShapeDtypeStruct((N, C), jnp.float32),
    )(x2, tile_W, tile_b, g_tile, b_tile)


def _finish_body(tl, pt, tr, ptk, cu, ro, g, b, o):
    o[:, 0, :, :] = tl[...] + pt[...]
    x = tr[...]
    m = jnp.mean(x, axis=-1, keepdims=True)
    v = jnp.mean((x - m) ** 2, axis=-1, keepdims=True)
    o[:, 1, :, :] = (x - m) / jnp.sqrt(v + EPS) * g[...][None, None, :] \
        + b[...][None, None, :] + ptk[...]
    o[:, 2, :, :] = cu[...]
    o[:, 3, :, :] = ro[...]


def _finish(tile_ln, pe_tile, tok_raw, pe_tok, cur_e, ro_e,
            g_token, b_token):
    BB = 128
    grid = (B // BB,)
    spec3 = pl.BlockSpec((BB, L, C), lambda i: (i, 0, 0))
    spec1 = pl.BlockSpec((C,), lambda i: (0,))
    out4 = pl.pallas_call(
        _finish_body,
        grid=grid,
        in_specs=[spec3, spec3, spec3, spec3, spec3, spec3, spec1, spec1],
        out_specs=pl.BlockSpec((BB, 4, L, C), lambda i: (i, 0, 0, 0)),
        out_shape=jax.ShapeDtypeStruct((B, 4, L, C), jnp.float32),
    )(tile_ln, pe_tile, tok_raw, pe_tok, cur_e, ro_e, g_token, b_token)
    return out4.reshape(B, 4 * L, C)


def kernel(tile_x, tile_yx, tile_t, token_x, token_t, cursor_fine_yxp,
           cursor_coarse_yx, cursor_t, readout_x, readout_t,
           tile_W, tile_b, token_table, cursor_fine_table, readout_table,
           spatial_pe, temporal_pe,
           g_tile, b_tile, g_token, b_token, g_cursor, b_cursor,
           g_readout, b_readout, g_spatial, b_spatial, g_temporal,
           b_temporal):
    sg, tg, cg, rg = _norm_tables(
        spatial_pe, temporal_pe, cursor_fine_table, readout_table,
        g_spatial, b_spatial, g_temporal, b_temporal,
        g_cursor, b_cursor, g_readout, b_readout)

    def ix(a):
        a = a.astype(jnp.int32).reshape(NW, NCHUNK, CH)
        a = jnp.pad(a, ((0, 0), (0, NCP - NCHUNK), (0, 0)))
        return a.reshape(NW * NCP, CH)

    idx_list = [ix(a) for a in (tile_yx, tile_t, token_x, token_t,
                                cursor_fine_yxp, cursor_coarse_yx,
                                cursor_t, readout_x, readout_t)]
    pe_tile, tok_raw, pe_tok, cur_e, ro_e = _sc_gather(
        sg, tg, cg, rg, token_table, idx_list)

    tile_ln = _tile_matmul(tile_x.reshape(N, TPIX), tile_W, tile_b,
                           g_tile, b_tile)

    r3 = lambda a: a.reshape(B, L, C)
    return _finish(r3(tile_ln), r3(pe_tile), r3(tok_raw), r3(pe_tok),
                   r3(cur_e), r3(ro_e), g_token, b_token)


# speedup vs baseline: 2.1062x; 1.3733x over previous
import functools

import jax
import jax.numpy as jnp
from jax import lax
from jax.experimental import pallas as pl
from jax.experimental.pallas import tpu as pltpu
from jax.experimental.pallas import tpu_sc as plsc

B, L, C = 1024, 20, 128
N = B * L
TPIX = 768
NC, NS = 2, 16
NW = NC * NS
PW = N // NW
CH = 128
NCHUNK = PW // CH
NCP = 8
EPS = 1e-5


def _norm_tables_body(sp, te, cf, ro, gs, bs, gt, bt, gc, bc, gr, br,
                      osp, ote, ocf, oro):
    def ln(x_ref, g_ref, b_ref, o_ref):
        x = x_ref[...]
        m = jnp.mean(x, axis=-1, keepdims=True)
        v = jnp.mean((x - m) ** 2, axis=-1, keepdims=True)
        o_ref[...] = (x - m) / jnp.sqrt(v + EPS) * g_ref[...][None, :] \
            + b_ref[...][None, :]
    ln(sp, gs, bs, osp)
    ln(te, gt, bt, ote)
    ln(cf, gc, bc, ocf)
    ln(ro, gr, br, oro)


def _norm_tables(spatial_pe, temporal_pe, cursor_fine_table, readout_table,
                 g_spatial, b_spatial, g_temporal, b_temporal,
                 g_cursor, b_cursor, g_readout, b_readout):
    outs = [jax.ShapeDtypeStruct(t.shape, jnp.float32)
            for t in (spatial_pe, temporal_pe, cursor_fine_table,
                      readout_table)]
    return pl.pallas_call(_norm_tables_body, out_shape=outs)(
        spatial_pe, temporal_pe, cursor_fine_table, readout_table,
        g_spatial, b_spatial, g_temporal, b_temporal,
        g_cursor, b_cursor, g_readout, b_readout)


def _sc_gather_body(sg, tg, cg, rg, tok,
                    ix_tile_yx, ix_tile_t, ix_token_x, ix_token_t,
                    ix_fine, ix_coarse, ix_cur_t, ix_ro_x, ix_ro_t,
                    pe_tile, tok_raw, pe_tok, cur_e, ro_e,
                    v_tile_yx, v_tile_t, v_token_x, v_token_t,
                    v_fine, v_coarse, v_cur_t, v_ro_x, v_ro_t,
                    b1, b2, b3, b4, b5, b6,
                    s1, s2, s3, s4, s5, s6, ss):
    wid = lax.axis_index("s") * NC + lax.axis_index("c")
    ibase = wid * NCP

    for hbm, vm in ((ix_tile_yx, v_tile_yx), (ix_tile_t, v_tile_t),
                    (ix_token_x, v_token_x), (ix_token_t, v_token_t),
                    (ix_fine, v_fine), (ix_coarse, v_coarse),
                    (ix_cur_t, v_cur_t), (ix_ro_x, v_ro_x),
                    (ix_ro_t, v_ro_t)):
        pltpu.sync_copy(hbm.at[pl.ds(ibase, NCP)], vm)

    def acc2(dst, src):
        def row(i, _):
            for c in range(8):
                sl = (i, pl.ds(c * 16, 16))
                plsc.addupdate(dst.at[sl], src[sl])
            return 0
        lax.fori_loop(0, CH, row, 0)

    def chunk(j, carry):
        out_off = wid * PW + j * CH

        def gth(tbl, vm, buf, sem):
            return pltpu.async_copy(tbl.at[vm.at[j]], buf, sem)

        def sto(buf, out_ref):
            return pltpu.async_copy(buf, out_ref.at[pl.ds(out_off, CH)], ss)

        c1 = gth(sg, v_tile_yx, b1, s1)
        c2 = gth(tg, v_tile_t, b2, s2)
        c3 = gth(tok, v_token_x, b3, s3)
        c4 = gth(tg, v_token_t, b4, s4)
        c5 = gth(cg, v_fine, b5, s5)
        c6 = gth(sg, v_coarse, b6, s6)
        c1.wait()
        c2.wait()
        acc2(b1, b2)
        st1 = sto(b1, pe_tile)
        c7 = gth(tg, v_cur_t, b2, s2)
        c3.wait()
        st2 = sto(b3, tok_raw)
        c4.wait()
        st3 = sto(b4, pe_tok)
        c5.wait()
        c6.wait()
        acc2(b5, b6)
        c8 = gth(rg, v_ro_x, b6, s6)
        c7.wait()
        acc2(b5, b2)
        st4 = sto(b5, cur_e)
        c9 = gth(tg, v_ro_t, b2, s2)
        c8.wait()
        c9.wait()
        acc2(b6, b2)
        st5 = sto(b6, ro_e)
        st1.wait()
        st2.wait()
        st3.wait()
        st4.wait()
        st5.wait()
        return carry

    lax.fori_loop(0, NCHUNK, chunk, 0)


def _sc_gather(sg, tg, cg, rg, tok, idx_list):
    mesh = plsc.VectorSubcoreMesh(core_axis_name="c", subcore_axis_name="s")
    f32 = jnp.float32
    out_type = [jax.ShapeDtypeStruct((N, C), f32) for _ in range(5)]
    idx_vmem = [pltpu.VMEM((NCP, CH), jnp.int32) for _ in range(9)]
    scratch = idx_vmem + [pltpu.VMEM((CH, C), f32) for _ in range(6)] + [
        pltpu.SemaphoreType.DMA for _ in range(7)]
    run = pl.kernel(_sc_gather_body, out_type=out_type, mesh=mesh,
                    scratch_types=scratch)
    return run(sg, tg, cg, rg, tok, *idx_list)


MB = 128
MR = MB * L


def _tile_matmul_body(x_ref, w_ref, bias_ref, g_ref, b_ref, o_ref):
    x = x_ref[...].reshape(MR, TPIX)
    y = jnp.dot(x, w_ref[...], preferred_element_type=jnp.float32)
    y = y + bias_ref[...][None, :]
    m = jnp.mean(y, axis=-1, keepdims=True)
    v = jnp.mean((y - m) ** 2, axis=-1, keepdims=True)
    o_ref[...] = (y - m) / jnp.sqrt(v + EPS) * g_ref[...][None, :] \
        + b_ref[...][None, :]


def _tile_matmul(tile_x, tile_W, tile_b, g_tile, b_tile):
    grid = (B // MB,)
    return pl.pallas_call(
        _tile_matmul_body,
        grid=grid,
        in_specs=[
            pl.BlockSpec((MB, L, TPIX), lambda i: (i, 0, 0)),
            pl.BlockSpec((TPIX, C), lambda i: (0, 0)),
            pl.BlockSpec((C,), lambda i: (0,)),
            pl.BlockSpec((C,), lambda i: (0,)),
            pl.BlockSpec((C,), lambda i: (0,)),
        ],
        out_specs=pl.BlockSpec((MR, C), lambda i: (i, 0)),
        out_shape=jax.ShapeDtypeStruct((N, C), jnp.float32),
    )(tile_x, tile_W, tile_b, g_tile, b_tile)


FB = 32
FR = FB * L


def _finish_body(tl, pt, tr, ptk, cu, ro, g, b, o):
    r3 = lambda v: v.reshape(FB, L, C)
    tile = r3(tl[...] + pt[...])
    x = tr[...]
    m = jnp.mean(x, axis=-1, keepdims=True)
    v = jnp.mean((x - m) ** 2, axis=-1, keepdims=True)
    token = r3((x - m) / jnp.sqrt(v + EPS) * g[...][None, :]
               + b[...][None, :] + ptk[...])
    cat = jnp.concatenate([tile, token, r3(cu[...]), r3(ro[...])], axis=1)
    o[...] = cat.reshape(FB * 4 * L, C)


def _finish(tile_ln, pe_tile, tok_raw, pe_tok, cur_e, ro_e,
            g_token, b_token):
    grid = (B // FB,)
    spec2 = pl.BlockSpec((FR, C), lambda i: (i, 0))
    spec1 = pl.BlockSpec((C,), lambda i: (0,))
    out2 = pl.pallas_call(
        _finish_body,
        grid=grid,
        in_specs=[spec2, spec2, spec2, spec2, spec2, spec2, spec1, spec1],
        out_specs=pl.BlockSpec((FB * 4 * L, C), lambda i: (i, 0)),
        out_shape=jax.ShapeDtypeStruct((B * 4 * L, C), jnp.float32),
    )(tile_ln, pe_tile, tok_raw, pe_tok, cur_e, ro_e, g_token, b_token)
    return out2.reshape(B, 4 * L, C)


def kernel(tile_x, tile_yx, tile_t, token_x, token_t, cursor_fine_yxp,
           cursor_coarse_yx, cursor_t, readout_x, readout_t,
           tile_W, tile_b, token_table, cursor_fine_table, readout_table,
           spatial_pe, temporal_pe,
           g_tile, b_tile, g_token, b_token, g_cursor, b_cursor,
           g_readout, b_readout, g_spatial, b_spatial, g_temporal,
           b_temporal):
    sg, tg, cg, rg = _norm_tables(
        spatial_pe, temporal_pe, cursor_fine_table, readout_table,
        g_spatial, b_spatial, g_temporal, b_temporal,
        g_cursor, b_cursor, g_readout, b_readout)

    def ix(a):
        a = a.astype(jnp.int32).reshape(NW, NCHUNK, CH)
        a = jnp.pad(a, ((0, 0), (0, NCP - NCHUNK), (0, 0)))
        return a.reshape(NW * NCP, CH)

    idx_list = [ix(a) for a in (tile_yx, tile_t, token_x, token_t,
                                cursor_fine_yxp, cursor_coarse_yx,
                                cursor_t, readout_x, readout_t)]
    pe_tile, tok_raw, pe_tok, cur_e, ro_e = _sc_gather(
        sg, tg, cg, rg, token_table, idx_list)

    tile_ln = _tile_matmul(tile_x, tile_W, tile_b, g_tile, b_tile)

    return _finish(tile_ln, pe_tile, tok_raw, pe_tok,
                   cur_e, ro_e, g_token, b_token)
